# baseline (speedup 1.0000x reference)
import functools

import jax
import jax.numpy as jnp
from jax import lax
from jax.experimental import pallas as pl
from jax.experimental.pallas import tpu as pltpu

B, H, W = 32, 14, 14
DIM, HEADS, MLP_DIM = 384, 12, 1536
E, TOPK = 8, 2
HEAD_DIM = DIM // HEADS
N = H * W


IMGS = 16


def _attn_body(x_ref, g1_ref, be1_ref, Wq_ref, bq_ref, Wk_ref, bk_ref,
               Wv_ref, bv_ref, Wo_ref, bo_ref, g2_ref, be2_ref, Wg_ref,
               bg_ref, xnew_ref, nx_ref, ap_ref, ti_ref, tp_ref):
    for img in range(IMGS):
        _attn_one(img, x_ref, g1_ref, be1_ref, Wq_ref, bq_ref, Wk_ref,
                  bk_ref, Wv_ref, bv_ref, Wo_ref, bo_ref, g2_ref, be2_ref,
                  Wg_ref, bg_ref, xnew_ref, nx_ref, ap_ref, ti_ref, tp_ref)


def _attn_one(img, x_ref, g1_ref, be1_ref, Wq_ref, bq_ref, Wk_ref, bk_ref,
              Wv_ref, bv_ref, Wo_ref, bo_ref, g2_ref, be2_ref, Wg_ref,
              bg_ref, xnew_ref, nx_ref, ap_ref, ti_ref, tp_ref):
    xb = x_ref[img]

    mu = jnp.mean(xb, axis=-1, keepdims=True)
    var = jnp.mean((xb - mu) ** 2, axis=-1, keepdims=True)
    n1 = (xb - mu) / jnp.sqrt(var + 1e-5) * g1_ref[0] + be1_ref[0]

    q = jnp.dot(n1, Wq_ref[...], preferred_element_type=jnp.float32) + bq_ref[0]
    k = jnp.dot(n1, Wk_ref[...], preferred_element_type=jnp.float32) + bk_ref[0]
    v = jnp.dot(n1, Wv_ref[...], preferred_element_type=jnp.float32) + bv_ref[0]

    ri = lax.broadcasted_iota(jnp.int32, (N, N), 0) // W
    ci = lax.broadcasted_iota(jnp.int32, (N, N), 1) // W
    row_mask = ri == ci

    outs = []
    for h in range(HEADS):
        sl = slice(h * HEAD_DIM, (h + 1) * HEAD_DIM)
        s = lax.dot_general(q[:, sl], k[:, sl], (((1,), (1,)), ((), ())),
                            preferred_element_type=jnp.float32)
        s = s / (HEAD_DIM ** 0.5)
        s = jnp.where(row_mask, s, -1e30)
        m = jnp.max(s, axis=-1, keepdims=True)
        e = jnp.exp(s - m)
        p = e / jnp.sum(e, axis=-1, keepdims=True)
        outs.append(jnp.dot(p, v[:, sl], preferred_element_type=jnp.float32))
    o = jnp.concatenate(outs, axis=1)
    o = jnp.dot(o, Wo_ref[...], preferred_element_type=jnp.float32) + bo_ref[0]
    xn = xb + o
    xnew_ref[img] = xn

    mu2 = jnp.mean(xn, axis=-1, keepdims=True)
    var2 = jnp.mean((xn - mu2) ** 2, axis=-1, keepdims=True)
    nx = (xn - mu2) / jnp.sqrt(var2 + 1e-5) * g2_ref[0] + be2_ref[0]
    nx_ref[img] = nx

    pooled = jnp.mean(nx, axis=0, keepdims=True)
    logits = jnp.dot(pooled, Wg_ref[...],
                     preferred_element_type=jnp.float32) + bg_ref[0]
    lm = jnp.max(logits, axis=-1, keepdims=True)
    ex = jnp.exp(logits - lm)
    probs = ex / jnp.sum(ex, axis=-1, keepdims=True)
    ap_ref[img] = probs

    col = lax.broadcasted_iota(jnp.int32, (1, E), 1)
    m1 = jnp.max(probs, axis=-1, keepdims=True)
    i1 = jnp.min(jnp.where(probs == m1, col, E), axis=-1, keepdims=True)
    pm = jnp.where(col == i1, -1.0, probs)
    m2 = jnp.max(pm, axis=-1, keepdims=True)
    i2 = jnp.min(jnp.where(pm == m2, col, E), axis=-1, keepdims=True)
    den = m1 + m2 + 1e-8
    tp_ref[img] = jnp.concatenate([m1 / den, m2 / den], axis=1)
    ti_ref[img] = jnp.concatenate([i1, i2], axis=1)


def _moe_body(ti_ref, tp_ref, nx_ref, xnew_ref, W1_ref, b1_ref, W2_ref,
              b2_ref, out_ref):
    i = pl.program_id(0)
    j = pl.program_id(1)
    w = tp_ref[i * TOPK + j]
    h = jnp.dot(nx_ref[0].astype(jnp.bfloat16), W1_ref[0],
                preferred_element_type=jnp.float32) + b1_ref[0]
    h = 0.5 * h * (1.0 + lax.erf(h * (2.0 ** -0.5)))
    eo = jnp.dot(h.astype(jnp.bfloat16), W2_ref[0],
                 preferred_element_type=jnp.float32) + b2_ref[0]

    @pl.when(j == 0)
    def _():
        out_ref[0] = xnew_ref[0] + w * eo

    @pl.when(j == 1)
    def _():
        out_ref[0] = out_ref[0] + w * eo


def kernel(x, g1, be1, Wq, bq, Wk, bk, Wv, bv, Wo, bo, g2, be2, Wg, bg,
           W1, b1, W2, b2):
    xf = x.reshape(B, N, DIM)
    r2 = lambda a: a.reshape(1, -1)

    const2 = lambda shape: pl.BlockSpec(shape, lambda b: (0, 0))
    attn_out = pl.pallas_call(
        _attn_body,
        grid=(B // IMGS,),
        in_specs=[
            pl.BlockSpec((IMGS, N, DIM), lambda b: (b, 0, 0)),
            const2((1, DIM)), const2((1, DIM)),
            const2((DIM, DIM)), const2((1, DIM)),
            const2((DIM, DIM)), const2((1, DIM)),
            const2((DIM, DIM)), const2((1, DIM)),
            const2((DIM, DIM)), const2((1, DIM)),
            const2((1, DIM)), const2((1, DIM)),
            const2((DIM, E)), const2((1, E)),
        ],
        out_specs=[
            pl.BlockSpec((IMGS, N, DIM), lambda b: (b, 0, 0)),
            pl.BlockSpec((IMGS, N, DIM), lambda b: (b, 0, 0)),
            pl.BlockSpec((IMGS, 1, E), lambda b: (b, 0, 0)),
            pl.BlockSpec((IMGS, 1, TOPK), lambda b: (b, 0, 0)),
            pl.BlockSpec((IMGS, 1, TOPK), lambda b: (b, 0, 0)),
        ],
        out_shape=[
            jax.ShapeDtypeStruct((B, N, DIM), jnp.float32),
            jax.ShapeDtypeStruct((B, N, DIM), jnp.float32),
            jax.ShapeDtypeStruct((B, 1, E), jnp.float32),
            jax.ShapeDtypeStruct((B, 1, TOPK), jnp.int32),
            jax.ShapeDtypeStruct((B, 1, TOPK), jnp.float32),
        ],
        compiler_params=pltpu.CompilerParams(
            dimension_semantics=("parallel",)),
    )(xf, r2(g1), r2(be1), Wq, r2(bq), Wk, r2(bk), Wv, r2(bv), Wo, r2(bo),
      r2(g2), r2(be2), Wg, r2(bg))

    xnew, nx, ap3, ti3, tp3 = attn_out
    all_probs = ap3.reshape(B, E)
    ti = ti3.reshape(B, TOPK)
    ti_flat = ti.reshape(B * TOPK)
    tp_flat = tp3.reshape(B * TOPK)

    grid_spec = pltpu.PrefetchScalarGridSpec(
        num_scalar_prefetch=2,
        grid=(B, TOPK),
        in_specs=[
            pl.BlockSpec((1, N, DIM), lambda i, j, ti_s, tp_s: (i, 0, 0)),
            pl.BlockSpec((1, N, DIM), lambda i, j, ti_s, tp_s: (i, 0, 0)),
            pl.BlockSpec((1, DIM, MLP_DIM),
                         lambda i, j, ti_s, tp_s: (ti_s[i * TOPK + j], 0, 0)),
            pl.BlockSpec((1, 1, MLP_DIM),
                         lambda i, j, ti_s, tp_s: (ti_s[i * TOPK + j], 0, 0)),
            pl.BlockSpec((1, MLP_DIM, DIM),
                         lambda i, j, ti_s, tp_s: (ti_s[i * TOPK + j], 0, 0)),
            pl.BlockSpec((1, 1, DIM),
                         lambda i, j, ti_s, tp_s: (ti_s[i * TOPK + j], 0, 0)),
        ],
        out_specs=pl.BlockSpec((1, N, DIM), lambda i, j, ti_s, tp_s: (i, 0, 0)),
    )
    out = pl.pallas_call(
        _moe_body,
        grid_spec=grid_spec,
        out_shape=jax.ShapeDtypeStruct((B, N, DIM), jnp.float32),
        compiler_params=pltpu.CompilerParams(
            dimension_semantics=("parallel", "arbitrary")),
    )(ti_flat, tp_flat, nx, xnew, W1.astype(jnp.bfloat16),
      b1.reshape(E, 1, MLP_DIM), W2.astype(jnp.bfloat16),
      b2.reshape(E, 1, DIM))

    return (out.reshape(B, H, W, DIM), all_probs, ti)


# speedup vs baseline: 1.1031x; 1.1031x over previous
import functools

import jax
import jax.numpy as jnp
from jax import lax
from jax.experimental import pallas as pl
from jax.experimental.pallas import tpu as pltpu
from jax.experimental.pallas import tpu_sc as plsc

B, H, W = 32, 14, 14
DIM, HEADS, MLP_DIM = 384, 12, 1536
E, TOPK = 8, 2
HEAD_DIM = DIM // HEADS
N = H * W


IMGS = 8


def _attn_body(x_ref, g1_ref, be1_ref, Wq_ref, bq_ref, Wk_ref, bk_ref,
               Wv_ref, bv_ref, Wo_ref, bo_ref, g2_ref, be2_ref, Wg_ref,
               bg_ref, xnew_ref, nx_ref, ap_ref):
    for img in range(IMGS):
        _attn_one(img, x_ref, g1_ref, be1_ref, Wq_ref, bq_ref, Wk_ref,
                  bk_ref, Wv_ref, bv_ref, Wo_ref, bo_ref, g2_ref, be2_ref,
                  Wg_ref, bg_ref, xnew_ref, nx_ref, ap_ref)


def _attn_one(img, x_ref, g1_ref, be1_ref, Wq_ref, bq_ref, Wk_ref, bk_ref,
              Wv_ref, bv_ref, Wo_ref, bo_ref, g2_ref, be2_ref, Wg_ref,
              bg_ref, xnew_ref, nx_ref, ap_ref):
    xb = x_ref[img]

    mu = jnp.mean(xb, axis=-1, keepdims=True)
    var = jnp.mean((xb - mu) ** 2, axis=-1, keepdims=True)
    n1 = (xb - mu) / jnp.sqrt(var + 1e-5) * g1_ref[0] + be1_ref[0]

    q = jnp.dot(n1, Wq_ref[...], preferred_element_type=jnp.float32) + bq_ref[0]
    k = jnp.dot(n1, Wk_ref[...], preferred_element_type=jnp.float32) + bk_ref[0]
    v = jnp.dot(n1, Wv_ref[...], preferred_element_type=jnp.float32) + bv_ref[0]

    ri = lax.broadcasted_iota(jnp.int32, (N, N), 0) // W
    ci = lax.broadcasted_iota(jnp.int32, (N, N), 1) // W
    row_mask = ri == ci

    outs = []
    for h in range(HEADS):
        sl = slice(h * HEAD_DIM, (h + 1) * HEAD_DIM)
        s = lax.dot_general(q[:, sl], k[:, sl], (((1,), (1,)), ((), ())),
                            preferred_element_type=jnp.float32)
        s = s / (HEAD_DIM ** 0.5)
        s = jnp.where(row_mask, s, -1e30)
        m = jnp.max(s, axis=-1, keepdims=True)
        e = jnp.exp(s - m)
        p = e / jnp.sum(e, axis=-1, keepdims=True)
        outs.append(jnp.dot(p, v[:, sl], preferred_element_type=jnp.float32))
    o = jnp.concatenate(outs, axis=1)
    o = jnp.dot(o, Wo_ref[...], preferred_element_type=jnp.float32) + bo_ref[0]
    xn = xb + o
    xnew_ref[img] = xn

    mu2 = jnp.mean(xn, axis=-1, keepdims=True)
    var2 = jnp.mean((xn - mu2) ** 2, axis=-1, keepdims=True)
    nx = (xn - mu2) / jnp.sqrt(var2 + 1e-5) * g2_ref[0] + be2_ref[0]
    nx_ref[img] = nx

    pooled = jnp.mean(nx, axis=0, keepdims=True)
    logits = jnp.dot(pooled, Wg_ref[...],
                     preferred_element_type=jnp.float32) + bg_ref[0]
    lm = jnp.max(logits, axis=-1, keepdims=True)
    ex = jnp.exp(logits - lm)
    probs = ex / jnp.sum(ex, axis=-1, keepdims=True)
    ap_ref[img] = probs


def _router_sc_body(ap_ref, ti_ref, tp_ref, p_v, ti_v, tp_v):
    cid = lax.axis_index("c")
    sid = lax.axis_index("s")

    @pl.when((cid == 0) & (sid == 0))
    def _():
        pltpu.sync_copy(ap_ref, p_v)
        for c in range(B // 16):
            sl = pl.ds(c * 16, 16)
            p = [p_v[e, sl] for e in range(E)]
            m1 = p[0]
            for e in range(1, E):
                m1 = jnp.maximum(m1, p[e])
            i1 = jnp.full((16,), E, jnp.int32)
            for e in range(E - 1, -1, -1):
                i1 = jnp.where(p[e] == m1, e, i1)
            pm = [jnp.where(i1 == e, -1.0, p[e]) for e in range(E)]
            m2 = pm[0]
            for e in range(1, E):
                m2 = jnp.maximum(m2, pm[e])
            i2 = jnp.full((16,), E, jnp.int32)
            for e in range(E - 1, -1, -1):
                i2 = jnp.where(pm[e] == m2, e, i2)
            den = m1 + m2 + 1e-8
            ti_v[0, sl] = i1
            ti_v[1, sl] = i2
            tp_v[0, sl] = m1 / den
            tp_v[1, sl] = m2 / den
        pltpu.sync_copy(ti_v, ti_ref)
        pltpu.sync_copy(tp_v, tp_ref)


def _router_sc(ap_T):
    run = pl.kernel(
        _router_sc_body,
        mesh=plsc.VectorSubcoreMesh(core_axis_name="c", subcore_axis_name="s"),
        out_type=[
            jax.ShapeDtypeStruct((TOPK, B), jnp.int32),
            jax.ShapeDtypeStruct((TOPK, B), jnp.float32),
        ],
        scratch_types=[
            pltpu.VMEM((E, B), jnp.float32),
            pltpu.VMEM((TOPK, B), jnp.int32),
            pltpu.VMEM((TOPK, B), jnp.float32),
        ],
    )
    return run(ap_T)


def _moe_body(ti_ref, tp_ref, nx_ref, xnew_ref, W1_ref, b1_ref, W2_ref,
              b2_ref, out_ref):
    i = pl.program_id(0)
    j = pl.program_id(1)
    w = tp_ref[i * TOPK + j]
    h = jnp.dot(nx_ref[0].astype(jnp.bfloat16), W1_ref[0],
                preferred_element_type=jnp.float32) + b1_ref[0]
    h = 0.5 * h * (1.0 + lax.erf(h * (2.0 ** -0.5)))
    eo = jnp.dot(h.astype(jnp.bfloat16), W2_ref[0],
                 preferred_element_type=jnp.float32) + b2_ref[0]

    @pl.when(j == 0)
    def _():
        out_ref[0] = xnew_ref[0] + w * eo

    @pl.when(j == 1)
    def _():
        out_ref[0] = out_ref[0] + w * eo


def kernel(x, g1, be1, Wq, bq, Wk, bk, Wv, bv, Wo, bo, g2, be2, Wg, bg,
           W1, b1, W2, b2):
    xf = x.reshape(B, N, DIM)
    r2 = lambda a: a.reshape(1, -1)

    const2 = lambda shape: pl.BlockSpec(shape, lambda b: (0, 0))
    attn_out = pl.pallas_call(
        _attn_body,
        grid=(B // IMGS,),
        in_specs=[
            pl.BlockSpec((IMGS, N, DIM), lambda b: (b, 0, 0)),
            const2((1, DIM)), const2((1, DIM)),
            const2((DIM, DIM)), const2((1, DIM)),
            const2((DIM, DIM)), const2((1, DIM)),
            const2((DIM, DIM)), const2((1, DIM)),
            const2((DIM, DIM)), const2((1, DIM)),
            const2((1, DIM)), const2((1, DIM)),
            const2((DIM, E)), const2((1, E)),
        ],
        out_specs=[
            pl.BlockSpec((IMGS, N, DIM), lambda b: (b, 0, 0)),
            pl.BlockSpec((IMGS, N, DIM), lambda b: (b, 0, 0)),
            pl.BlockSpec((IMGS, 1, E), lambda b: (b, 0, 0)),
        ],
        out_shape=[
            jax.ShapeDtypeStruct((B, N, DIM), jnp.float32),
            jax.ShapeDtypeStruct((B, N, DIM), jnp.float32),
            jax.ShapeDtypeStruct((B, 1, E), jnp.float32),
        ],
        compiler_params=pltpu.CompilerParams(
            dimension_semantics=("parallel",)),
    )(xf, r2(g1), r2(be1), Wq, r2(bq), Wk, r2(bk), Wv, r2(bv), Wo, r2(bo),
      r2(g2), r2(be2), Wg, r2(bg))

    xnew, nx, ap3 = attn_out
    all_probs = ap3.reshape(B, E)
    ti_T, tp_T = _router_sc(all_probs.T)
    ti = ti_T.T
    ti_flat = ti.reshape(B * TOPK)
    tp_flat = tp_T.T.reshape(B * TOPK)

    grid_spec = pltpu.PrefetchScalarGridSpec(
        num_scalar_prefetch=2,
        grid=(B, TOPK),
        in_specs=[
            pl.BlockSpec((1, N, DIM), lambda i, j, ti_s, tp_s: (i, 0, 0)),
            pl.BlockSpec((1, N, DIM), lambda i, j, ti_s, tp_s: (i, 0, 0)),
            pl.BlockSpec((1, DIM, MLP_DIM),
                         lambda i, j, ti_s, tp_s: (ti_s[i * TOPK + j], 0, 0)),
            pl.BlockSpec((1, 1, MLP_DIM),
                         lambda i, j, ti_s, tp_s: (ti_s[i * TOPK + j], 0, 0)),
            pl.BlockSpec((1, MLP_DIM, DIM),
                         lambda i, j, ti_s, tp_s: (ti_s[i * TOPK + j], 0, 0)),
            pl.BlockSpec((1, 1, DIM),
                         lambda i, j, ti_s, tp_s: (ti_s[i * TOPK + j], 0, 0)),
        ],
        out_specs=pl.BlockSpec((1, N, DIM), lambda i, j, ti_s, tp_s: (i, 0, 0)),
    )
    out = pl.pallas_call(
        _moe_body,
        grid_spec=grid_spec,
        out_shape=jax.ShapeDtypeStruct((B, N, DIM), jnp.float32),
        compiler_params=pltpu.CompilerParams(
            dimension_semantics=("parallel", "arbitrary")),
    )(ti_flat, tp_flat, nx, xnew, W1.astype(jnp.bfloat16),
      b1.reshape(E, 1, MLP_DIM), W2.astype(jnp.bfloat16),
      b2.reshape(E, 1, DIM))

    return (out.reshape(B, H, W, DIM), all_probs, ti)


# speedup vs baseline: 1.1138x; 1.0098x over previous
import functools

import jax
import jax.numpy as jnp
from jax import lax
from jax.experimental import pallas as pl
from jax.experimental.pallas import tpu as pltpu
from jax.experimental.pallas import tpu_sc as plsc

B, H, W = 32, 14, 14
DIM, HEADS, MLP_DIM = 384, 12, 1536
E, TOPK = 8, 2
HEAD_DIM = DIM // HEADS
N = H * W


IMGS = 8


def _attn_body(x_ref, g1_ref, be1_ref, Wq_ref, bq_ref, Wk_ref, bk_ref,
               Wv_ref, bv_ref, Wo_ref, bo_ref, g2_ref, be2_ref, Wg_ref,
               bg_ref, xnew_ref, nx_ref, ap_ref):
    for img in range(IMGS):
        _attn_one(img, x_ref, g1_ref, be1_ref, Wq_ref, bq_ref, Wk_ref,
                  bk_ref, Wv_ref, bv_ref, Wo_ref, bo_ref, g2_ref, be2_ref,
                  Wg_ref, bg_ref, xnew_ref, nx_ref, ap_ref)


def _attn_one(img, x_ref, g1_ref, be1_ref, Wq_ref, bq_ref, Wk_ref, bk_ref,
              Wv_ref, bv_ref, Wo_ref, bo_ref, g2_ref, be2_ref, Wg_ref,
              bg_ref, xnew_ref, nx_ref, ap_ref):
    xb = x_ref[img]

    mu = jnp.mean(xb, axis=-1, keepdims=True)
    var = jnp.mean((xb - mu) ** 2, axis=-1, keepdims=True)
    n1 = (xb - mu) / jnp.sqrt(var + 1e-5) * g1_ref[0] + be1_ref[0]

    q = jnp.dot(n1, Wq_ref[...], preferred_element_type=jnp.float32) + bq_ref[0]
    k = jnp.dot(n1, Wk_ref[...], preferred_element_type=jnp.float32) + bk_ref[0]
    v = jnp.dot(n1, Wv_ref[...], preferred_element_type=jnp.float32) + bv_ref[0]

    ri = lax.broadcasted_iota(jnp.int32, (N, N), 0) // W
    ci = lax.broadcasted_iota(jnp.int32, (N, N), 1) // W
    row_mask = ri == ci

    outs = []
    for h in range(HEADS):
        sl = slice(h * HEAD_DIM, (h + 1) * HEAD_DIM)
        s = lax.dot_general(q[:, sl], k[:, sl], (((1,), (1,)), ((), ())),
                            preferred_element_type=jnp.float32)
        s = s / (HEAD_DIM ** 0.5)
        s = jnp.where(row_mask, s, -1e30)
        m = jnp.max(s, axis=-1, keepdims=True)
        e = jnp.exp(s - m)
        p = e / jnp.sum(e, axis=-1, keepdims=True)
        outs.append(jnp.dot(p, v[:, sl], preferred_element_type=jnp.float32))
    o = jnp.concatenate(outs, axis=1)
    o = jnp.dot(o, Wo_ref[...], preferred_element_type=jnp.float32) + bo_ref[0]
    xn = xb + o
    xnew_ref[img] = xn

    mu2 = jnp.mean(xn, axis=-1, keepdims=True)
    var2 = jnp.mean((xn - mu2) ** 2, axis=-1, keepdims=True)
    nx = (xn - mu2) / jnp.sqrt(var2 + 1e-5) * g2_ref[0] + be2_ref[0]
    nx_ref[img] = nx

    pooled = jnp.mean(nx, axis=0, keepdims=True)
    logits = jnp.dot(pooled, Wg_ref[...],
                     preferred_element_type=jnp.float32) + bg_ref[0]
    lm = jnp.max(logits, axis=-1, keepdims=True)
    ex = jnp.exp(logits - lm)
    probs = ex / jnp.sum(ex, axis=-1, keepdims=True)
    ap_ref[img] = probs


def _router_sc_body(ap_ref, ti_ref, tp_ref, p_v, ti_v, tp_v):
    cid = lax.axis_index("c")
    sid = lax.axis_index("s")

    @pl.when((cid == 0) & (sid == 0))
    def _():
        pltpu.sync_copy(ap_ref, p_v)
        for c in range(B // 16):
            sl = pl.ds(c * 16, 16)
            p = [p_v[e, sl] for e in range(E)]
            m1 = p[0]
            for e in range(1, E):
                m1 = jnp.maximum(m1, p[e])
            i1 = jnp.full((16,), E, jnp.int32)
            for e in range(E - 1, -1, -1):
                i1 = jnp.where(p[e] == m1, e, i1)
            pm = [jnp.where(i1 == e, -1.0, p[e]) for e in range(E)]
            m2 = pm[0]
            for e in range(1, E):
                m2 = jnp.maximum(m2, pm[e])
            i2 = jnp.full((16,), E, jnp.int32)
            for e in range(E - 1, -1, -1):
                i2 = jnp.where(pm[e] == m2, e, i2)
            den = m1 + m2 + 1e-8
            ti_v[0, sl] = i1
            ti_v[1, sl] = i2
            tp_v[0, sl] = m1 / den
            tp_v[1, sl] = m2 / den
        pltpu.sync_copy(ti_v, ti_ref)
        pltpu.sync_copy(tp_v, tp_ref)


def _router_sc(ap_T):
    run = pl.kernel(
        _router_sc_body,
        mesh=plsc.VectorSubcoreMesh(core_axis_name="c", subcore_axis_name="s"),
        out_type=[
            jax.ShapeDtypeStruct((TOPK, B), jnp.int32),
            jax.ShapeDtypeStruct((TOPK, B), jnp.float32),
        ],
        scratch_types=[
            pltpu.VMEM((E, B), jnp.float32),
            pltpu.VMEM((TOPK, B), jnp.int32),
            pltpu.VMEM((TOPK, B), jnp.float32),
        ],
    )
    return run(ap_T)


HB = B // 2
HP = HB * TOPK


def _dispatch_body(ti_col_ref, ti_row_ref, tp_col_ref, e_ref, b_ref, t_ref):
    pc = lax.broadcasted_iota(jnp.int32, (TOPK * B, TOPK * B), 0)
    pr = lax.broadcasted_iota(jnp.int32, (TOPK * B, TOPK * B), 1)
    kc = ti_col_ref[...] + E * (pc >= HP).astype(jnp.int32)
    kr = ti_row_ref[...] + E * (pr >= HP).astype(jnp.int32)
    before = (kr < kc) | ((kr == kc) & (pr < pc))
    rank = jnp.sum(before.astype(jnp.int32), axis=1, keepdims=True)
    s_row = lax.broadcasted_iota(jnp.int32, (1, TOPK * B), 1)
    oh = (rank == s_row).astype(jnp.float32)
    pos_col = lax.broadcasted_iota(jnp.int32, (TOPK * B, 1), 0).astype(
        jnp.float32)
    order = jnp.sum(oh * pos_col, axis=0, keepdims=True).astype(jnp.int32)
    e_ref[...] = jnp.sum(oh * ti_col_ref[...].astype(jnp.float32),
                         axis=0, keepdims=True).astype(jnp.int32)
    b_ref[...] = (order // TOPK) % HB
    t_ref[...] = jnp.sum(oh * tp_col_ref[...], axis=0, keepdims=True)


def _moe_body(e_ref, b_ref, t_ref, nx_ref, xnew_ref, W1_ref, b1_ref, W2_ref,
              b2_ref, out_ref, acc_ref):
    c = pl.program_id(0)
    s = pl.program_id(1)
    idx = c * HP + s
    b = b_ref[idx]
    w = t_ref[idx]

    @pl.when(s == 0)
    def _():
        acc_ref[...] = jnp.zeros((HB, N, DIM), jnp.float32)

    h = jnp.dot(nx_ref[b].astype(jnp.bfloat16), W1_ref[0],
                preferred_element_type=jnp.float32) + b1_ref[0]
    h = 0.5 * h * (1.0 + lax.erf(h * (2.0 ** -0.5)))
    eo = jnp.dot(h.astype(jnp.bfloat16), W2_ref[0],
                 preferred_element_type=jnp.float32) + b2_ref[0]
    acc_ref[b] = acc_ref[b] + w * eo

    @pl.when(s == HP - 1)
    def _():
        out_ref[...] = xnew_ref[...] + acc_ref[...]


def kernel(x, g1, be1, Wq, bq, Wk, bk, Wv, bv, Wo, bo, g2, be2, Wg, bg,
           W1, b1, W2, b2):
    xf = x.reshape(B, N, DIM)
    r2 = lambda a: a.reshape(1, -1)

    const2 = lambda shape: pl.BlockSpec(shape, lambda b: (0, 0))
    attn_out = pl.pallas_call(
        _attn_body,
        grid=(B // IMGS,),
        in_specs=[
            pl.BlockSpec((IMGS, N, DIM), lambda b: (b, 0, 0)),
            const2((1, DIM)), const2((1, DIM)),
            const2((DIM, DIM)), const2((1, DIM)),
            const2((DIM, DIM)), const2((1, DIM)),
            const2((DIM, DIM)), const2((1, DIM)),
            const2((DIM, DIM)), const2((1, DIM)),
            const2((1, DIM)), const2((1, DIM)),
            const2((DIM, E)), const2((1, E)),
        ],
        out_specs=[
            pl.BlockSpec((IMGS, N, DIM), lambda b: (b, 0, 0)),
            pl.BlockSpec((IMGS, N, DIM), lambda b: (b, 0, 0)),
            pl.BlockSpec((IMGS, 1, E), lambda b: (b, 0, 0)),
        ],
        out_shape=[
            jax.ShapeDtypeStruct((B, N, DIM), jnp.float32),
            jax.ShapeDtypeStruct((B, N, DIM), jnp.float32),
            jax.ShapeDtypeStruct((B, 1, E), jnp.float32),
        ],
        compiler_params=pltpu.CompilerParams(
            dimension_semantics=("parallel",)),
    )(xf, r2(g1), r2(be1), Wq, r2(bq), Wk, r2(bk), Wv, r2(bv), Wo, r2(bo),
      r2(g2), r2(be2), Wg, r2(bg))

    xnew, nx, ap3 = attn_out
    all_probs = ap3.reshape(B, E)
    ti_T, tp_T = _router_sc(all_probs.T)
    ti = ti_T.T
    ti_flat = ti.reshape(B * TOPK)
    tp_flat = tp_T.T.reshape(B * TOPK)

    full_row = pl.BlockSpec((1, TOPK * B), lambda: (0, 0))
    full_col = pl.BlockSpec((TOPK * B, 1), lambda: (0, 0))
    e_s, b_s, t_s = pl.pallas_call(
        _dispatch_body,
        in_specs=[full_col, full_row, full_col],
        out_specs=[full_row, full_row, full_row],
        out_shape=[
            jax.ShapeDtypeStruct((1, TOPK * B), jnp.int32),
            jax.ShapeDtypeStruct((1, TOPK * B), jnp.int32),
            jax.ShapeDtypeStruct((1, TOPK * B), jnp.float32),
        ],
    )(ti_flat.reshape(TOPK * B, 1), ti_flat.reshape(1, TOPK * B),
      tp_flat.reshape(TOPK * B, 1))

    grid_spec = pltpu.PrefetchScalarGridSpec(
        num_scalar_prefetch=3,
        grid=(2, HP),
        in_specs=[
            pl.BlockSpec((HB, N, DIM), lambda c, s, e_r, b_r, t_r: (c, 0, 0)),
            pl.BlockSpec((HB, N, DIM), lambda c, s, e_r, b_r, t_r: (c, 0, 0)),
            pl.BlockSpec((1, DIM, MLP_DIM),
                         lambda c, s, e_r, b_r, t_r: (e_r[c * HP + s], 0, 0)),
            pl.BlockSpec((1, 1, MLP_DIM),
                         lambda c, s, e_r, b_r, t_r: (e_r[c * HP + s], 0, 0)),
            pl.BlockSpec((1, MLP_DIM, DIM),
                         lambda c, s, e_r, b_r, t_r: (e_r[c * HP + s], 0, 0)),
            pl.BlockSpec((1, 1, DIM),
                         lambda c, s, e_r, b_r, t_r: (e_r[c * HP + s], 0, 0)),
        ],
        out_specs=pl.BlockSpec((HB, N, DIM),
                               lambda c, s, e_r, b_r, t_r: (c, 0, 0)),
        scratch_shapes=[pltpu.VMEM((HB, N, DIM), jnp.float32)],
    )
    out = pl.pallas_call(
        _moe_body,
        grid_spec=grid_spec,
        out_shape=jax.ShapeDtypeStruct((B, N, DIM), jnp.float32),
        compiler_params=pltpu.CompilerParams(
            dimension_semantics=("parallel", "arbitrary")),
    )(e_s.reshape(TOPK * B), b_s.reshape(TOPK * B), t_s.reshape(TOPK * B),
      nx, xnew, W1.astype(jnp.bfloat16), b1.reshape(E, 1, MLP_DIM),
      W2.astype(jnp.bfloat16), b2.reshape(E, 1, DIM))

    return (out.reshape(B, H, W, DIM), all_probs, ti)
